# baseline (speedup 1.0000x reference)
import jax
import jax.numpy as jnp
from jax.experimental import pallas as pl


def _onepass_kernel(ft_ref, fb_ref, wq_ref, wk_ref, nbr_ref, out_ref):
    q = ft_ref[...] * wq_ref[...]
    kb = fb_ref[...] * wk_ref[...]
    g = jax.lax.dot_general(
        q, kb, (((1,), (0,)), ((), ())),
        preferred_element_type=jnp.float32,
    )
    s = jnp.exp(g)
    m_t = jnp.transpose(nbr_ref[...])
    masked = jnp.where(m_t != 0, s, 0.0)
    summed = jnp.sum(masked, axis=0)
    out_ref[...] = s / summed[None, :]


@jax.jit
def kernel(f, neighbors, wq, wk):
    size, n = f.shape
    b = 256
    grid = (pl.cdiv(n, b),)
    ft = f.T
    wq_r = wq.reshape(1, size)
    wk_c = wk.reshape(size, 1)

    out = pl.pallas_call(
        _onepass_kernel,
        grid=grid,
        in_specs=[
            pl.BlockSpec((n, size), lambda i: (0, 0)),
            pl.BlockSpec((size, b), lambda i: (0, i)),
            pl.BlockSpec((1, size), lambda i: (0, 0)),
            pl.BlockSpec((size, 1), lambda i: (0, 0)),
            pl.BlockSpec((b, n), lambda i: (i, 0)),
        ],
        out_specs=pl.BlockSpec((n, b), lambda i: (0, i)),
        out_shape=jax.ShapeDtypeStruct((n, n), jnp.float32),
    )(ft, f, wq_r, wk_c, neighbors)
    return out
